# Initial kernel scaffold; baseline (speedup 1.0000x reference)
import jax
import jax.numpy as jnp
from jax import lax
from jax.experimental import pallas as pl
from jax.experimental.pallas import tpu as pltpu
from jax.experimental.pallas import tpu_sc as plsc

N = 10000
D = 128
E = 160000

NC = 2
NS = 16
NW = NC * NS
EPW = E // NW
CH = 128
EPW_PAD = 5120
NCHUNK = EPW_PAD // CH
DUMMY = N
NPAD = 10240
RPS = NPAD // NS
ZR = 128

_NEG = -1e9


def _sc_body(h_hbm, src_hbm, dst_hbm, ones_hbm, zeros_hbm,
             agg_out, deg_out,
             src_v, dst_v, rows_v, ones_v, zbuf_v, zdeg_v,
             agg_sh, deg_sh, sem):
  cid = lax.axis_index("c")
  sid = lax.axis_index("s")
  wid = cid * NS + sid

  pltpu.sync_copy(zeros_hbm, zbuf_v)
  pltpu.sync_copy(zeros_hbm.at[pl.ds(0, ZR), pl.ds(0, 16)], zdeg_v)
  pltpu.sync_copy(ones_hbm, ones_v)

  base = sid * RPS
  for b in range(RPS // ZR):
    sl = pl.ds(base + b * ZR, ZR)
    pltpu.sync_copy(zbuf_v, agg_sh.at[sl])
    pltpu.sync_copy(zdeg_v, deg_sh.at[sl])

  pltpu.sync_copy(src_hbm.at[wid], src_v)
  pltpu.sync_copy(dst_hbm.at[wid], dst_v)

  plsc.subcore_barrier()

  @pl.loop(0, NCHUNK)
  def _(j):
    pltpu.async_copy(h_hbm.at[src_v.at[j]], rows_v, sem).wait()
    pltpu.sync_copy(rows_v, agg_sh.at[dst_v.at[j]], add=True)
    pltpu.sync_copy(ones_v, deg_sh.at[dst_v.at[j]], add=True)

  plsc.subcore_barrier()

  for b in range(RPS // ZR):
    sl = pl.ds(base + b * ZR, ZR)
    pltpu.sync_copy(agg_sh.at[sl], agg_out.at[cid].at[sl])
    pltpu.sync_copy(deg_sh.at[sl], deg_out.at[cid].at[sl])


_sc_scatter = pl.kernel(
    _sc_body,
    out_type=(
        jax.ShapeDtypeStruct((NC, NPAD, D), jnp.float32),
        jax.ShapeDtypeStruct((NC, NPAD, 16), jnp.float32),
    ),
    mesh=plsc.VectorSubcoreMesh(core_axis_name="c", subcore_axis_name="s"),
    scratch_types=[
        pltpu.VMEM((NCHUNK, CH), jnp.int32),
        pltpu.VMEM((NCHUNK, CH), jnp.int32),
        pltpu.VMEM((CH, D), jnp.float32),
        pltpu.VMEM((CH, 16), jnp.float32),
        pltpu.VMEM((ZR, D), jnp.float32),
        pltpu.VMEM((ZR, 16), jnp.float32),
        pltpu.VMEM_SHARED((NPAD, D), jnp.float32),
        pltpu.VMEM_SHARED((NPAD, 16), jnp.float32),
        pltpu.SemaphoreType.DMA,
    ],
)


def _col0(r):
  return lax.broadcasted_iota(jnp.int32, (r, D), 1) == 0


def _proj_rows(y, m0):
  s2 = jnp.sum(jnp.where(m0, 0.0, y * y), axis=1, keepdims=True)
  return jnp.where(m0, jnp.sqrt(1.0 + s2), y)


def _linT(x, w):
  return lax.dot_general(x, w, (((1,), (1,)), ((), ())),
                         precision=lax.Precision.HIGHEST,
                         preferred_element_type=jnp.float32)


def _lorentz_inner(a, b, m0):
  p = a * b
  return jnp.sum(jnp.where(m0, -p, p), axis=1, keepdims=True)


def _lnormalize(c, m0):
  inner = _lorentz_inner(c, c, m0)
  return c / jnp.sqrt(jnp.clip(-inner, 1e-6, None))


def _tc_a_body(x_ref, wagg_ref, x0_ref, h2_ref):
  r = x_ref.shape[0]
  m0 = _col0(r)
  x0 = _proj_rows(x_ref[...], m0)
  h2 = _proj_rows(_linT(x0, wagg_ref[...]), m0)
  x0_ref[...] = x0
  h2_ref[...] = h2


def _tc_b_body(aggp_ref, degp_ref, h_ref, x0_ref, wagg_ref,
               z_ref, mf_ref, h1_ref):
  r = h_ref.shape[0]
  m0 = _col0(r)
  parts = aggp_ref[...]
  agg = parts[0] + parts[1]
  degp = degp_ref[...]
  deg = jnp.sum(degp[0] + degp[1], axis=1, keepdims=True)
  upd = deg > 0.0
  nrm = _lnormalize(agg, m0)
  h = h_ref[...]
  z = jnp.where(upd, nrm, h)
  x1 = jnp.where(upd, nrm, x0_ref[...])
  z_ref[...] = z
  mf_ref[...] = jnp.where(jnp.broadcast_to(upd, (r, D)), 1.0, 0.0)
  h1_ref[...] = _proj_rows(_linT(x1, wagg_ref[...]), m0)


def _tc_c_body(aggp_ref, degp_ref, h1_ref, x0_ref, z2_ref, mf_ref,
               wq_ref, wk_ref, wv_ref, out_ref):
  r = h1_ref.shape[0]
  m0 = _col0(r)
  parts = aggp_ref[...]
  agg = parts[0] + parts[1]
  degp = degp_ref[...]
  deg = jnp.sum(degp[0] + degp[1], axis=1, keepdims=True)
  upd1 = deg > 0.0
  out1 = jnp.where(upd1, _lnormalize(agg, m0), h1_ref[...])

  x0 = x0_ref[...]
  z2 = z2_ref[...]
  upd2 = mf_ref[...][:, 0:1] > 0.5

  wq = wq_ref[...]
  wk = wk_ref[...]
  wv = wv_ref[...]
  q = _proj_rows(_linT(x0, wq), m0)
  k0 = _proj_rows(_linT(x0, wk), m0)
  k1 = _proj_rows(_linT(z2, wk), m0)
  k2 = _proj_rows(_linT(out1, wk), m0)
  v0 = _proj_rows(_linT(x0, wv), m0)
  v1 = _proj_rows(_linT(z2, wv), m0)
  v2 = _proj_rows(_linT(out1, wv), m0)

  s0 = _lorentz_inner(q, k0, m0)
  s1 = jnp.where(upd2, _lorentz_inner(q, k1, m0), _NEG)
  s2 = jnp.where(upd1, _lorentz_inner(q, k2, m0), _NEG)
  m = jnp.maximum(jnp.maximum(s0, s1), s2)
  e0 = jnp.exp(s0 - m)
  e1 = jnp.exp(s1 - m)
  e2 = jnp.exp(s2 - m)
  c = (e0 * v0 + e1 * v1 + e2 * v2) / (e0 + e1 + e2)
  out_ref[...] = _lnormalize(c, m0)


_R = 1000
_G = N // _R

_rows = lambda i: (i, 0)
_rows3 = lambda i: (0, i, 0)
_whole = lambda i: (0, 0)

_bs_rows = pl.BlockSpec((_R, D), _rows)
_bs_w = pl.BlockSpec((D, D), _whole)
_bs_aggp = pl.BlockSpec((NC, _R, D), _rows3)
_bs_degp = pl.BlockSpec((NC, _R, 16), _rows3)

_tc_a = pl.pallas_call(
    _tc_a_body,
    grid=(_G,),
    in_specs=[_bs_rows, _bs_w],
    out_specs=[_bs_rows, _bs_rows],
    out_shape=(jax.ShapeDtypeStruct((N, D), jnp.float32),
               jax.ShapeDtypeStruct((N, D), jnp.float32)),
)

_tc_b = pl.pallas_call(
    _tc_b_body,
    grid=(_G,),
    in_specs=[_bs_aggp, _bs_degp, _bs_rows, _bs_rows, _bs_w],
    out_specs=[_bs_rows, _bs_rows, _bs_rows],
    out_shape=(jax.ShapeDtypeStruct((N, D), jnp.float32),
               jax.ShapeDtypeStruct((N, D), jnp.float32),
               jax.ShapeDtypeStruct((N, D), jnp.float32)),
)

_tc_c = pl.pallas_call(
    _tc_c_body,
    grid=(_G,),
    in_specs=[_bs_aggp, _bs_degp, _bs_rows, _bs_rows, _bs_rows, _bs_rows,
              _bs_w, _bs_w, _bs_w],
    out_specs=_bs_rows,
    out_shape=jax.ShapeDtypeStruct((N, D), jnp.float32),
)


def _prep_edges(edges):
  src = edges[0].astype(jnp.int32).reshape(NW, EPW)
  dst = edges[1].astype(jnp.int32).reshape(NW, EPW)
  pad = EPW_PAD - EPW
  src = jnp.pad(src, ((0, 0), (0, pad)), constant_values=0)
  dst = jnp.pad(dst, ((0, 0), (0, pad)), constant_values=DUMMY)
  return (src.reshape(NW, NCHUNK, CH), dst.reshape(NW, NCHUNK, CH))


@jax.jit
def kernel(x_H, edge_index_l1, edge_index_l2, Wq, Wk, Wv, W_agg):
  src2, dst2 = _prep_edges(edge_index_l2)
  src1, dst1 = _prep_edges(edge_index_l1)
  ones = jnp.ones((CH, 16), jnp.float32)
  zeros = jnp.zeros((ZR, D), jnp.float32)

  x0, h2 = _tc_a(x_H, W_agg)
  agg2p, deg2p = _sc_scatter(h2, src2, dst2, ones, zeros)
  z2, m2f, h1 = _tc_b(agg2p, deg2p, h2, x0, W_agg)
  agg1p, deg1p = _sc_scatter(h1, src1, dst1, ones, zeros)
  return _tc_c(agg1p, deg1p, h1, x0, z2, m2f, Wq, Wk, Wv)


# speedup vs baseline: 3.1334x; 3.1334x over previous
import functools

import jax
import jax.numpy as jnp
from jax import lax
from jax.experimental import pallas as pl
from jax.experimental.pallas import tpu as pltpu
from jax.experimental.pallas import tpu_sc as plsc

N = 10000
D = 128
E = 160000

NC = 2
NS = 16
NW = NC * NS
EPW = E // NW
CH = 64
EPW_PAD = 5120
NCHUNK = EPW_PAD // CH
DUMMY = N
NPAD = 10240
RPS = NPAD // NS
ZR = 128

_NEG = -1e9


SBC = 8
NSB = NCHUNK // SBC


def _sc_body(h_hbm, src_hbm, dst_hbm, zeros_hbm,
             agg_out,
             src_v, dst_v, rows_v,
             agg_sh, sem):
  cid = lax.axis_index("c")
  sid = lax.axis_index("s")
  wid = cid * NS + sid

  base = sid * RPS
  for b in range(RPS // ZR):
    pltpu.sync_copy(zeros_hbm, agg_sh.at[pl.ds(base + b * ZR, ZR)])

  plsc.subcore_barrier()

  @pl.loop(0, NSB)
  def _(sb):
    pltpu.sync_copy(
        src_hbm.at[pl.ds(wid * EPW_PAD + sb * (SBC * CH), SBC * CH)], src_v)
    pltpu.sync_copy(dst_hbm.at[wid * NSB + sb], dst_v)

    @pl.loop(0, SBC)
    def _(j):
      pltpu.async_copy(
          h_hbm.at[src_v.at[pl.ds(j * CH, CH)]], rows_v, sem).wait()
      pltpu.sync_copy(rows_v, agg_sh.at[dst_v.at[j]], add=True)

  plsc.subcore_barrier()

  for b in range(RPS // ZR):
    off = cid * NPAD + base + b * ZR
    pltpu.sync_copy(agg_sh.at[pl.ds(base + b * ZR, ZR)],
                    agg_out.at[pl.ds(off, ZR)])


@functools.cache
def _get_sc_scatter():
  return pl.kernel(
    _sc_body,
    out_type=jax.ShapeDtypeStruct((NC * NPAD, D), jnp.float32),
    mesh=plsc.VectorSubcoreMesh(core_axis_name="c", subcore_axis_name="s",
                                num_cores=NC, num_subcores=NS),
    scratch_types=[
        pltpu.VMEM((SBC * CH,), jnp.int32),
        pltpu.VMEM((SBC, CH), jnp.int32),
        pltpu.VMEM((CH, D), jnp.float32),
        pltpu.VMEM_SHARED((NPAD, D), jnp.float32),
        pltpu.SemaphoreType.DMA,
    ],
  )


def _col0(r):
  return lax.broadcasted_iota(jnp.int32, (r, D), 1) == 0


def _proj_rows(y, m0):
  s2 = jnp.sum(jnp.where(m0, 0.0, y * y), axis=1, keepdims=True)
  return jnp.where(m0, jnp.sqrt(1.0 + s2), y)


def _linT(x, w):
  return lax.dot_general(x, w, (((1,), (1,)), ((), ())),
                         precision=lax.Precision.HIGHEST,
                         preferred_element_type=jnp.float32)


def _lorentz_inner(a, b, m0):
  p = a * b
  return jnp.sum(jnp.where(m0, -p, p), axis=1, keepdims=True)


def _lnormalize(c, m0):
  inner = _lorentz_inner(c, c, m0)
  return c / jnp.sqrt(jnp.clip(-inner, 1e-6, None))


def _tc_a_body(x_ref, wagg_ref, x0_ref, h2_ref):
  r = x_ref.shape[0]
  m0 = _col0(r)
  x0 = _proj_rows(x_ref[...], m0)
  h2 = _proj_rows(_linT(x0, wagg_ref[...]), m0)
  x0_ref[...] = x0
  h2_ref[...] = h2


def _tc_b_body(aggp_ref, h_ref, x0_ref, wagg_ref,
               z_ref, mf_ref, h1_ref):
  r = h_ref.shape[0]
  m0 = _col0(r)
  parts = aggp_ref[...]
  agg = parts[0] + parts[1]
  upd = agg[:, 0:1] > 0.5
  nrm = _lnormalize(agg, m0)
  h = h_ref[...]
  z = jnp.where(upd, nrm, h)
  x1 = jnp.where(upd, nrm, x0_ref[...])
  z_ref[...] = z
  mf_ref[...] = jnp.where(jnp.broadcast_to(upd, (r, D)), 1.0, 0.0)
  h1_ref[...] = _proj_rows(_linT(x1, wagg_ref[...]), m0)


def _tc_c_body(aggp_ref, h1_ref, x0_ref, z2_ref, mf_ref,
               wq_ref, wk_ref, wv_ref, out_ref):
  r = h1_ref.shape[0]
  m0 = _col0(r)
  parts = aggp_ref[...]
  agg = parts[0] + parts[1]
  upd1 = agg[:, 0:1] > 0.5
  out1 = jnp.where(upd1, _lnormalize(agg, m0), h1_ref[...])

  x0 = x0_ref[...]
  z2 = z2_ref[...]
  upd2 = mf_ref[...][:, 0:1] > 0.5

  wq = wq_ref[...]
  wk = wk_ref[...]
  wv = wv_ref[...]
  q = _proj_rows(_linT(x0, wq), m0)
  k0 = _proj_rows(_linT(x0, wk), m0)
  k1 = _proj_rows(_linT(z2, wk), m0)
  k2 = _proj_rows(_linT(out1, wk), m0)
  v0 = _proj_rows(_linT(x0, wv), m0)
  v1 = _proj_rows(_linT(z2, wv), m0)
  v2 = _proj_rows(_linT(out1, wv), m0)

  s0 = _lorentz_inner(q, k0, m0)
  s1 = jnp.where(upd2, _lorentz_inner(q, k1, m0), _NEG)
  s2 = jnp.where(upd1, _lorentz_inner(q, k2, m0), _NEG)
  m = jnp.maximum(jnp.maximum(s0, s1), s2)
  e0 = jnp.exp(s0 - m)
  e1 = jnp.exp(s1 - m)
  e2 = jnp.exp(s2 - m)
  c = (e0 * v0 + e1 * v1 + e2 * v2) / (e0 + e1 + e2)
  out_ref[...] = _lnormalize(c, m0)


_R = 1000
_G = N // _R

_rows = lambda i: (i, 0)
_rows3 = lambda i: (0, i, 0)
_whole = lambda i: (0, 0)

_bs_rows = pl.BlockSpec((_R, D), _rows)
_bs_w = pl.BlockSpec((D, D), _whole)
_bs_aggp = pl.BlockSpec((NC, _R, D), _rows3)

_tc_a = pl.pallas_call(
    _tc_a_body,
    grid=(_G,),
    in_specs=[_bs_rows, _bs_w],
    out_specs=[_bs_rows, _bs_rows],
    out_shape=(jax.ShapeDtypeStruct((N, D), jnp.float32),
               jax.ShapeDtypeStruct((N, D), jnp.float32)),
)

_tc_b = pl.pallas_call(
    _tc_b_body,
    grid=(_G,),
    in_specs=[_bs_aggp, _bs_rows, _bs_rows, _bs_w],
    out_specs=[_bs_rows, _bs_rows, _bs_rows],
    out_shape=(jax.ShapeDtypeStruct((N, D), jnp.float32),
               jax.ShapeDtypeStruct((N, D), jnp.float32),
               jax.ShapeDtypeStruct((N, D), jnp.float32)),
)

_tc_c = pl.pallas_call(
    _tc_c_body,
    grid=(_G,),
    in_specs=[_bs_aggp, _bs_rows, _bs_rows, _bs_rows, _bs_rows,
              _bs_w, _bs_w, _bs_w],
    out_specs=_bs_rows,
    out_shape=jax.ShapeDtypeStruct((N, D), jnp.float32),
)


def _prep_edges(edges):
  src = edges[0].astype(jnp.int32).reshape(NW, EPW)
  dst = edges[1].astype(jnp.int32).reshape(NW, EPW)
  pad = EPW_PAD - EPW
  src = jnp.pad(src, ((0, 0), (0, pad)), constant_values=0)
  dst = jnp.pad(dst, ((0, 0), (0, pad)), constant_values=DUMMY)
  return (src.reshape(NW * EPW_PAD), dst.reshape(NW * NSB, SBC, CH))


@jax.jit
def kernel(x_H, edge_index_l1, edge_index_l2, Wq, Wk, Wv, W_agg):
  src2, dst2 = _prep_edges(edge_index_l2)
  src1, dst1 = _prep_edges(edge_index_l1)
  zeros = jnp.zeros((ZR, D), jnp.float32)

  sc_scatter = _get_sc_scatter()
  x0, h2 = _tc_a(x_H, W_agg)
  agg2p = sc_scatter(h2, src2, dst2, zeros).reshape(NC, NPAD, D)
  z2, m2f, h1 = _tc_b(agg2p, h2, x0, W_agg)
  agg1p = sc_scatter(h1, src1, dst1, zeros).reshape(NC, NPAD, D)
  return _tc_c(agg1p, h1, x0, z2, m2f, Wq, Wk, Wv)


# speedup vs baseline: 3.4871x; 1.1129x over previous
import functools

import jax
import jax.numpy as jnp
from jax import lax
from jax.experimental import pallas as pl
from jax.experimental.pallas import tpu as pltpu
from jax.experimental.pallas import tpu_sc as plsc

N = 10000
D = 128
E = 160000

NC = 2
NS = 16
NW = NC * NS
EPW = E // NW
CH = 64
EPW_PAD = 5120
NCHUNK = EPW_PAD // CH
DUMMY = N
NPAD = 10240
RPS = NPAD // NS
ZR = 128

_NEG = -1e9


SBC = 8
NSB = NCHUNK // SBC


def _sc_body(h_hbm, src_hbm, dst_hbm, zeros_hbm,
             agg_out,
             src_v, dst_v, rows_v, rows2_v,
             agg_sh, sem, sem2):
  cid = lax.axis_index("c")
  sid = lax.axis_index("s")
  wid = cid * NS + sid

  base = sid * RPS
  for b in range(RPS // ZR):
    pltpu.sync_copy(zeros_hbm, agg_sh.at[pl.ds(base + b * ZR, ZR)])

  pltpu.sync_copy(src_hbm.at[pl.ds(wid * EPW_PAD, EPW_PAD)], src_v)
  pltpu.sync_copy(dst_hbm.at[wid], dst_v)

  plsc.subcore_barrier()

  def _gather_start(j, buf, s):
    return pltpu.async_copy(h_hbm.at[src_v.at[pl.ds(j * CH, CH)]], buf, s)

  def _gather_wait(j, buf, s):
    pltpu.make_async_copy(h_hbm.at[src_v.at[pl.ds(j * CH, CH)]], buf, s).wait()

  _gather_start(0, rows_v, sem)

  @pl.loop(0, NCHUNK // 2)
  def _(g):
    j0 = 2 * g
    _gather_wait(j0, rows_v, sem)
    _gather_start(j0 + 1, rows2_v, sem2)
    pltpu.sync_copy(rows_v, agg_sh.at[dst_v.at[j0]], add=True)
    _gather_wait(j0 + 1, rows2_v, sem2)

    @pl.when(g + 1 < NCHUNK // 2)
    def _():
      _gather_start(j0 + 2, rows_v, sem)

    pltpu.sync_copy(rows2_v, agg_sh.at[dst_v.at[j0 + 1]], add=True)

  plsc.subcore_barrier()

  for b in range(RPS // ZR):
    off = cid * NPAD + base + b * ZR
    pltpu.sync_copy(agg_sh.at[pl.ds(base + b * ZR, ZR)],
                    agg_out.at[pl.ds(off, ZR)])


@functools.cache
def _get_sc_scatter():
  return pl.kernel(
    _sc_body,
    out_type=jax.ShapeDtypeStruct((NC * NPAD, D), jnp.float32),
    mesh=plsc.VectorSubcoreMesh(core_axis_name="c", subcore_axis_name="s",
                                num_cores=NC, num_subcores=NS),
    scratch_types=[
        pltpu.VMEM((EPW_PAD,), jnp.int32),
        pltpu.VMEM((NCHUNK, CH), jnp.int32),
        pltpu.VMEM((CH, D), jnp.float32),
        pltpu.VMEM((CH, D), jnp.float32),
        pltpu.VMEM_SHARED((NPAD, D), jnp.float32),
        pltpu.SemaphoreType.DMA,
        pltpu.SemaphoreType.DMA,
    ],
  )


def _col0(r):
  return lax.broadcasted_iota(jnp.int32, (r, D), 1) == 0


def _proj_rows(y, m0):
  s2 = jnp.sum(jnp.where(m0, 0.0, y * y), axis=1, keepdims=True)
  return jnp.where(m0, jnp.sqrt(1.0 + s2), y)


def _linT(x, w):
  return lax.dot_general(x, w, (((1,), (1,)), ((), ())),
                         precision=lax.Precision.HIGHEST,
                         preferred_element_type=jnp.float32)


def _lorentz_inner(a, b, m0):
  p = a * b
  return jnp.sum(jnp.where(m0, -p, p), axis=1, keepdims=True)


def _lnormalize(c, m0):
  inner = _lorentz_inner(c, c, m0)
  return c / jnp.sqrt(jnp.clip(-inner, 1e-6, None))


def _tc_a_body(x_ref, wagg_ref, x0_ref, h2_ref):
  r = x_ref.shape[0]
  m0 = _col0(r)
  x0 = _proj_rows(x_ref[...], m0)
  h2 = _proj_rows(_linT(x0, wagg_ref[...]), m0)
  x0_ref[...] = x0
  h2_ref[...] = h2


def _tc_b_body(aggp_ref, h_ref, x0_ref, wagg_ref,
               z_ref, mf_ref, h1_ref):
  r = h_ref.shape[0]
  m0 = _col0(r)
  parts = aggp_ref[...]
  agg = parts[0] + parts[1]
  upd = agg[:, 0:1] > 0.5
  nrm = _lnormalize(agg, m0)
  h = h_ref[...]
  z = jnp.where(upd, nrm, h)
  x1 = jnp.where(upd, nrm, x0_ref[...])
  z_ref[...] = z
  mf_ref[...] = jnp.where(jnp.broadcast_to(upd, (r, D)), 1.0, 0.0)
  h1_ref[...] = _proj_rows(_linT(x1, wagg_ref[...]), m0)


def _tc_c_body(aggp_ref, h1_ref, x0_ref, z2_ref, mf_ref,
               wq_ref, wk_ref, wv_ref, out_ref):
  r = h1_ref.shape[0]
  m0 = _col0(r)
  parts = aggp_ref[...]
  agg = parts[0] + parts[1]
  upd1 = agg[:, 0:1] > 0.5
  out1 = jnp.where(upd1, _lnormalize(agg, m0), h1_ref[...])

  x0 = x0_ref[...]
  z2 = z2_ref[...]
  upd2 = mf_ref[...][:, 0:1] > 0.5

  wq = wq_ref[...]
  wk = wk_ref[...]
  wv = wv_ref[...]
  q = _proj_rows(_linT(x0, wq), m0)
  k0 = _proj_rows(_linT(x0, wk), m0)
  k1 = _proj_rows(_linT(z2, wk), m0)
  k2 = _proj_rows(_linT(out1, wk), m0)
  v0 = _proj_rows(_linT(x0, wv), m0)
  v1 = _proj_rows(_linT(z2, wv), m0)
  v2 = _proj_rows(_linT(out1, wv), m0)

  s0 = _lorentz_inner(q, k0, m0)
  s1 = jnp.where(upd2, _lorentz_inner(q, k1, m0), _NEG)
  s2 = jnp.where(upd1, _lorentz_inner(q, k2, m0), _NEG)
  m = jnp.maximum(jnp.maximum(s0, s1), s2)
  e0 = jnp.exp(s0 - m)
  e1 = jnp.exp(s1 - m)
  e2 = jnp.exp(s2 - m)
  c = (e0 * v0 + e1 * v1 + e2 * v2) / (e0 + e1 + e2)
  out_ref[...] = _lnormalize(c, m0)


_R = 1000
_G = N // _R

_rows = lambda i: (i, 0)
_rows3 = lambda i: (0, i, 0)
_whole = lambda i: (0, 0)

_bs_rows = pl.BlockSpec((_R, D), _rows)
_bs_w = pl.BlockSpec((D, D), _whole)
_bs_aggp = pl.BlockSpec((NC, _R, D), _rows3)

_tc_a = pl.pallas_call(
    _tc_a_body,
    grid=(_G,),
    in_specs=[_bs_rows, _bs_w],
    out_specs=[_bs_rows, _bs_rows],
    out_shape=(jax.ShapeDtypeStruct((N, D), jnp.float32),
               jax.ShapeDtypeStruct((N, D), jnp.float32)),
)

_tc_b = pl.pallas_call(
    _tc_b_body,
    grid=(_G,),
    in_specs=[_bs_aggp, _bs_rows, _bs_rows, _bs_w],
    out_specs=[_bs_rows, _bs_rows, _bs_rows],
    out_shape=(jax.ShapeDtypeStruct((N, D), jnp.float32),
               jax.ShapeDtypeStruct((N, D), jnp.float32),
               jax.ShapeDtypeStruct((N, D), jnp.float32)),
)

_tc_c = pl.pallas_call(
    _tc_c_body,
    grid=(_G,),
    in_specs=[_bs_aggp, _bs_rows, _bs_rows, _bs_rows, _bs_rows,
              _bs_w, _bs_w, _bs_w],
    out_specs=_bs_rows,
    out_shape=jax.ShapeDtypeStruct((N, D), jnp.float32),
)


def _prep_edges(edges):
  src = edges[0].astype(jnp.int32).reshape(NW, EPW)
  dst = edges[1].astype(jnp.int32).reshape(NW, EPW)
  pad = EPW_PAD - EPW
  src = jnp.pad(src, ((0, 0), (0, pad)), constant_values=0)
  dst = jnp.pad(dst, ((0, 0), (0, pad)), constant_values=DUMMY)
  return (src.reshape(NW * EPW_PAD), dst.reshape(NW, NCHUNK, CH))


@jax.jit
def kernel(x_H, edge_index_l1, edge_index_l2, Wq, Wk, Wv, W_agg):
  src2, dst2 = _prep_edges(edge_index_l2)
  src1, dst1 = _prep_edges(edge_index_l1)
  zeros = jnp.zeros((ZR, D), jnp.float32)

  sc_scatter = _get_sc_scatter()
  x0, h2 = _tc_a(x_H, W_agg)
  agg2p = sc_scatter(h2, src2, dst2, zeros).reshape(NC, NPAD, D)
  z2, m2f, h1 = _tc_b(agg2p, h2, x0, W_agg)
  agg1p = sc_scatter(h1, src1, dst1, zeros).reshape(NC, NPAD, D)
  return _tc_c(agg1p, h1, x0, z2, m2f, Wq, Wk, Wv)
